# Initial kernel scaffold; baseline (speedup 1.0000x reference)
import jax
import jax.numpy as jnp
from jax.experimental import pallas as pl


def kernel(x, edge_index, w1, b1, wg0, bg0, wg1, bg1, wg2, bg2, wc, bc):
    raise NotImplementedError("write your pallas kernel here")


# speedup vs baseline: 17.3005x; 17.3005x over previous
import functools

import jax
import jax.numpy as jnp
from jax import lax
from jax.experimental import pallas as pl
from jax.experimental.pallas import tpu as pltpu
from jax.experimental.pallas import tpu_sc as plsc

N = 10000
HID = 128
DOUT = 64
E = 320000
NPAD = 10112
NW = 32
NCH = 80
K = 128
EPAD = NW * NCH * K
RPT = NPAD // 16

_mesh = plsc.VectorSubcoreMesh(core_axis_name="c", subcore_axis_name="s")


_DEG_KW = dict(
    out_type=jax.ShapeDtypeStruct((2, NPAD, HID), jnp.float32),
    mesh=_mesh,
    scratch_types=[
        pltpu.VMEM((NCH, K), jnp.int32),
        pltpu.VMEM((K, HID), jnp.float32),
        pltpu.VMEM_SHARED((NPAD, HID), jnp.float32),
    ],
)


def _deg_body(dsti_hbm, zeros_hbm, ones_hbm, out_hbm, dstv, ones_v, acc):
    c = lax.axis_index("c")
    s = lax.axis_index("s")
    w = c * 16 + s
    base = s * RPT
    pltpu.sync_copy(dsti_hbm.at[w], dstv)
    pltpu.sync_copy(ones_hbm, ones_v)
    pltpu.sync_copy(zeros_hbm.at[pl.ds(base, RPT)], acc.at[pl.ds(base, RPT)])
    plsc.subcore_barrier()

    def _chunk(j, carry):
        pltpu.sync_copy(ones_v, acc.at[dstv.at[j]], add=True)
        return carry

    lax.fori_loop(0, NCH, _chunk, 0)
    plsc.subcore_barrier()
    pltpu.sync_copy(acc.at[pl.ds(base, RPT)], out_hbm.at[c, pl.ds(base, RPT)])


_SEG_KW = dict(
    out_type=jax.ShapeDtypeStruct((2, NPAD, HID), jnp.float32),
    mesh=_mesh,
    scratch_types=[
        pltpu.VMEM((NCH // 2, K), jnp.int32),
        pltpu.VMEM((NCH // 2, K), jnp.int32),
        pltpu.VMEM((K, HID), jnp.float32),
        pltpu.VMEM((K, HID), jnp.float32),
        pltpu.VMEM_SHARED((NPAD, HID), jnp.float32),
        pltpu.SemaphoreType.DMA,
        pltpu.SemaphoreType.DMA,
    ],
)


def _seg_body(hs_hbm, srci_hbm, dsti_hbm, zeros_hbm, out_hbm, srcv, dstv, buf0,
              buf1, acc, sem0, sem1):
    c = lax.axis_index("c")
    s = lax.axis_index("s")
    w = c * 16 + s
    half = NCH // 2
    base = s * RPT

    for hf in range(2):
        pltpu.sync_copy(srci_hbm.at[w, pl.ds(hf * half, half)], srcv)
        pltpu.sync_copy(dsti_hbm.at[w, pl.ds(hf * half, half)], dstv)

        if hf == 0:
            pltpu.sync_copy(zeros_hbm.at[pl.ds(base, RPT)],
                            acc.at[pl.ds(base, RPT)])
            plsc.subcore_barrier()

        def _chunk(j, carry):
            pltpu.sync_copy(hs_hbm.at[srcv.at[j]], buf0)
            pltpu.sync_copy(buf0, acc.at[dstv.at[j]], add=True)
            return carry

        lax.fori_loop(0, half, _chunk, 0)

    plsc.subcore_barrier()
    pltpu.sync_copy(acc.at[pl.ds(base, RPT)], out_hbm.at[c, pl.ds(base, RPT)])


_deg = pl.kernel(_deg_body, **_DEG_KW)
_seg = pl.kernel(_seg_body, **_SEG_KW)


_MB = 1000
_G = N // _MB


def _tc1_body(deg_ref, x_ref, w1_ref, b1_ref, wg0_ref, dinv_ref, hs0_ref):
    deg = 1.0 + deg_ref[0][:, 0:1] + deg_ref[1][:, 0:1]
    dinv = lax.rsqrt(deg)
    h = jnp.dot(x_ref[...], w1_ref[...],
                preferred_element_type=jnp.float32) + b1_ref[...]
    hs0_ref[...] = jnp.dot(h, wg0_ref[...],
                           preferred_element_type=jnp.float32) * dinv
    dinv_ref[...] = dinv


_tc1 = pl.pallas_call(
    _tc1_body,
    grid=(_G,),
    in_specs=[
        pl.BlockSpec((2, _MB, HID), lambda i: (0, i, 0)),
        pl.BlockSpec((_MB, HID), lambda i: (i, 0)),
        pl.BlockSpec((HID, HID), lambda i: (0, 0)),
        pl.BlockSpec((1, HID), lambda i: (0, 0)),
        pl.BlockSpec((HID, HID), lambda i: (0, 0)),
    ],
    out_specs=[
        pl.BlockSpec((_MB, 1), lambda i: (i, 0)),
        pl.BlockSpec((_MB, HID), lambda i: (i, 0)),
    ],
    out_shape=[
        jax.ShapeDtypeStruct((N, 1), jnp.float32),
        jax.ShapeDtypeStruct((N, HID), jnp.float32),
    ],
)


def _tcm_body(acc_ref, hs_ref, dinv_ref, b_ref, w_ref, j_ref, hsn_ref):
    dinv = dinv_ref[...]
    t = dinv * (acc_ref[0] + acc_ref[1] + hs_ref[...]) + b_ref[...]
    j = jnp.maximum(t, 0.0)
    j_ref[...] = j
    hsn_ref[...] = jnp.dot(j, w_ref[...],
                           preferred_element_type=jnp.float32) * dinv


_tcm = pl.pallas_call(
    _tcm_body,
    grid=(_G,),
    in_specs=[
        pl.BlockSpec((2, _MB, HID), lambda i: (0, i, 0)),
        pl.BlockSpec((_MB, HID), lambda i: (i, 0)),
        pl.BlockSpec((_MB, 1), lambda i: (i, 0)),
        pl.BlockSpec((1, HID), lambda i: (0, 0)),
        pl.BlockSpec((HID, HID), lambda i: (0, 0)),
    ],
    out_specs=[
        pl.BlockSpec((_MB, HID), lambda i: (i, 0)),
        pl.BlockSpec((_MB, HID), lambda i: (i, 0)),
    ],
    out_shape=[
        jax.ShapeDtypeStruct((N, HID), jnp.float32),
        jax.ShapeDtypeStruct((N, HID), jnp.float32),
    ],
)


def _tcm2_body(acc_ref, hs_ref, dinv_ref, b_ref, w_ref, jp_ref, s_ref,
               hsn_ref):
    dinv = dinv_ref[...]
    t = dinv * (acc_ref[0] + acc_ref[1] + hs_ref[...]) + b_ref[...]
    j = jnp.maximum(t, 0.0)
    s_ref[...] = jp_ref[...] + j
    hsn_ref[...] = jnp.dot(j, w_ref[...],
                           preferred_element_type=jnp.float32) * dinv


_tcm2 = pl.pallas_call(
    _tcm2_body,
    grid=(_G,),
    in_specs=[
        pl.BlockSpec((2, _MB, HID), lambda i: (0, i, 0)),
        pl.BlockSpec((_MB, HID), lambda i: (i, 0)),
        pl.BlockSpec((_MB, 1), lambda i: (i, 0)),
        pl.BlockSpec((1, HID), lambda i: (0, 0)),
        pl.BlockSpec((HID, HID), lambda i: (0, 0)),
        pl.BlockSpec((_MB, HID), lambda i: (i, 0)),
    ],
    out_specs=[
        pl.BlockSpec((_MB, HID), lambda i: (i, 0)),
        pl.BlockSpec((_MB, HID), lambda i: (i, 0)),
    ],
    out_shape=[
        jax.ShapeDtypeStruct((N, HID), jnp.float32),
        jax.ShapeDtypeStruct((N, HID), jnp.float32),
    ],
)


def _tcf_body(acc_ref, hs_ref, dinv_ref, b_ref, s_ref, wc_ref, bc_ref,
              out_ref):
    dinv = dinv_ref[...]
    t = dinv * (acc_ref[0] + acc_ref[1] + hs_ref[...]) + b_ref[...]
    j = jnp.maximum(t, 0.0)
    x5 = s_ref[...] + j
    out_ref[...] = jnp.dot(x5, wc_ref[...],
                           preferred_element_type=jnp.float32) + bc_ref[...]


_tcf = pl.pallas_call(
    _tcf_body,
    grid=(_G,),
    in_specs=[
        pl.BlockSpec((2, _MB, HID), lambda i: (0, i, 0)),
        pl.BlockSpec((_MB, HID), lambda i: (i, 0)),
        pl.BlockSpec((_MB, 1), lambda i: (i, 0)),
        pl.BlockSpec((1, HID), lambda i: (0, 0)),
        pl.BlockSpec((_MB, HID), lambda i: (i, 0)),
        pl.BlockSpec((HID, DOUT), lambda i: (0, 0)),
        pl.BlockSpec((1, DOUT), lambda i: (0, 0)),
    ],
    out_specs=pl.BlockSpec((_MB, DOUT), lambda i: (i, 0)),
    out_shape=jax.ShapeDtypeStruct((N, DOUT), jnp.float32),
)


def kernel(x, edge_index, w1, b1, wg0, bg0, wg1, bg1, wg2, bg2, wc, bc):
    ei = edge_index.astype(jnp.int32)
    pad = EPAD - E
    ar = jnp.arange(pad, dtype=jnp.int32)
    srcp = jnp.concatenate([ei[0], (ar * 37) % N]).reshape(NW, NCH, K)
    dstp = jnp.concatenate([ei[1], N + (ar % (NPAD - N))]).reshape(NW, NCH, K)

    zrows = jnp.zeros((NPAD, HID), jnp.float32)
    orows = jnp.ones((K, HID), jnp.float32)
    degp = _deg(dstp, zrows, orows)
    dinv, hs0 = _tc1(degp, x, w1, b1.reshape(1, HID), wg0)
    acc0 = _seg(hs0, srcp, dstp, zrows)
    j1, hs1 = _tcm(acc0, hs0, dinv, bg0.reshape(1, HID), wg1)
    acc1 = _seg(hs1, srcp, dstp, zrows)
    s2, hs2 = _tcm2(acc1, hs1, dinv, bg1.reshape(1, HID), wg2, j1)
    acc2 = _seg(hs2, srcp, dstp, zrows)
    return _tcf(acc2, hs2, dinv, bg2.reshape(1, HID), s2, wc,
                bc.reshape(1, DOUT))


# speedup vs baseline: 24.2686x; 1.4028x over previous
import functools

import jax
import jax.numpy as jnp
from jax import lax
from jax.experimental import pallas as pl
from jax.experimental.pallas import tpu as pltpu
from jax.experimental.pallas import tpu_sc as plsc

N = 10000
HID = 128
DOUT = 64
E = 320000
NPAD = 10112
NW = 32
NCH = 80
K = 128
EPAD = NW * NCH * K
RPT = NPAD // 16

_mesh = plsc.VectorSubcoreMesh(core_axis_name="c", subcore_axis_name="s")


_DEG_KW = dict(
    out_type=jax.ShapeDtypeStruct((2, NPAD, HID), jnp.float32),
    mesh=_mesh,
    scratch_types=[
        pltpu.VMEM((NCH, K), jnp.int32),
        pltpu.VMEM((K, HID), jnp.float32),
        pltpu.VMEM_SHARED((NPAD, HID), jnp.float32),
    ],
)


def _deg_body(dsti_hbm, zeros_hbm, ones_hbm, out_hbm, dstv, ones_v, acc):
    c = lax.axis_index("c")
    s = lax.axis_index("s")
    w = c * 16 + s
    base = s * RPT
    pltpu.sync_copy(dsti_hbm.at[w], dstv)
    pltpu.sync_copy(ones_hbm, ones_v)
    pltpu.sync_copy(zeros_hbm.at[pl.ds(base, RPT)], acc.at[pl.ds(base, RPT)])
    plsc.subcore_barrier()

    def _chunk(j, carry):
        pltpu.sync_copy(ones_v, acc.at[dstv.at[j]], add=True)
        return carry

    lax.fori_loop(0, NCH, _chunk, 0)
    plsc.subcore_barrier()
    pltpu.sync_copy(acc.at[pl.ds(base, RPT)], out_hbm.at[c, pl.ds(base, RPT)])


_SEG_KW = dict(
    out_type=jax.ShapeDtypeStruct((2, NPAD, HID), jnp.float32),
    mesh=_mesh,
    scratch_types=[
        pltpu.VMEM((NCH // 2, K), jnp.int32),
        pltpu.VMEM((NCH // 2, K), jnp.int32),
        pltpu.VMEM((K, HID), jnp.float32),
        pltpu.VMEM((K, HID), jnp.float32),
        pltpu.VMEM_SHARED((NPAD, HID), jnp.float32),
        pltpu.SemaphoreType.DMA,
        pltpu.SemaphoreType.DMA,
    ],
)


def _seg_body(hs_hbm, srci_hbm, dsti_hbm, zeros_hbm, out_hbm, srcv, dstv, buf0,
              buf1, acc, sem0, sem1):
    c = lax.axis_index("c")
    s = lax.axis_index("s")
    w = c * 16 + s
    half = NCH // 2
    base = s * RPT

    for hf in range(2):
        pltpu.sync_copy(srci_hbm.at[w, pl.ds(hf * half, half)], srcv)
        pltpu.sync_copy(dsti_hbm.at[w, pl.ds(hf * half, half)], dstv)

        if hf == 0:
            pltpu.sync_copy(zeros_hbm.at[pl.ds(base, RPT)],
                            acc.at[pl.ds(base, RPT)])
            plsc.subcore_barrier()

        pltpu.async_copy(hs_hbm.at[srcv.at[0]], buf0, sem0)
        pltpu.async_copy(hs_hbm.at[srcv.at[1]], buf1, sem1)

        def _pair(it, carry):
            j0 = it * 2
            j1 = j0 + 1
            pltpu.make_async_copy(hs_hbm.at[srcv.at[j0]], buf0, sem0).wait()
            pltpu.sync_copy(buf0, acc.at[dstv.at[j0]], add=True)

            @pl.when(j0 + 2 < half)
            def _():
                pltpu.async_copy(hs_hbm.at[srcv.at[j0 + 2]], buf0, sem0)

            pltpu.make_async_copy(hs_hbm.at[srcv.at[j1]], buf1, sem1).wait()
            pltpu.sync_copy(buf1, acc.at[dstv.at[j1]], add=True)

            @pl.when(j1 + 2 < half)
            def _():
                pltpu.async_copy(hs_hbm.at[srcv.at[j1 + 2]], buf1, sem1)

            return carry

        lax.fori_loop(0, half // 2, _pair, 0)

    plsc.subcore_barrier()
    pltpu.sync_copy(acc.at[pl.ds(base, RPT)], out_hbm.at[c, pl.ds(base, RPT)])


_deg = pl.kernel(_deg_body, **_DEG_KW)
_seg = pl.kernel(_seg_body, **_SEG_KW)


_MB = 1000
_G = N // _MB


def _tc1_body(deg_ref, x_ref, w1_ref, b1_ref, wg0_ref, dinv_ref, hs0_ref):
    deg = 1.0 + deg_ref[0][:, 0:1] + deg_ref[1][:, 0:1]
    dinv = lax.rsqrt(deg)
    h = jnp.dot(x_ref[...], w1_ref[...],
                preferred_element_type=jnp.float32) + b1_ref[...]
    hs0_ref[...] = jnp.dot(h, wg0_ref[...],
                           preferred_element_type=jnp.float32) * dinv
    dinv_ref[...] = dinv


_tc1 = pl.pallas_call(
    _tc1_body,
    grid=(_G,),
    in_specs=[
        pl.BlockSpec((2, _MB, HID), lambda i: (0, i, 0)),
        pl.BlockSpec((_MB, HID), lambda i: (i, 0)),
        pl.BlockSpec((HID, HID), lambda i: (0, 0)),
        pl.BlockSpec((1, HID), lambda i: (0, 0)),
        pl.BlockSpec((HID, HID), lambda i: (0, 0)),
    ],
    out_specs=[
        pl.BlockSpec((_MB, 1), lambda i: (i, 0)),
        pl.BlockSpec((_MB, HID), lambda i: (i, 0)),
    ],
    out_shape=[
        jax.ShapeDtypeStruct((N, 1), jnp.float32),
        jax.ShapeDtypeStruct((N, HID), jnp.float32),
    ],
)


def _tcm_body(acc_ref, hs_ref, dinv_ref, b_ref, w_ref, j_ref, hsn_ref):
    dinv = dinv_ref[...]
    t = dinv * (acc_ref[0] + acc_ref[1] + hs_ref[...]) + b_ref[...]
    j = jnp.maximum(t, 0.0)
    j_ref[...] = j
    hsn_ref[...] = jnp.dot(j, w_ref[...],
                           preferred_element_type=jnp.float32) * dinv


_tcm = pl.pallas_call(
    _tcm_body,
    grid=(_G,),
    in_specs=[
        pl.BlockSpec((2, _MB, HID), lambda i: (0, i, 0)),
        pl.BlockSpec((_MB, HID), lambda i: (i, 0)),
        pl.BlockSpec((_MB, 1), lambda i: (i, 0)),
        pl.BlockSpec((1, HID), lambda i: (0, 0)),
        pl.BlockSpec((HID, HID), lambda i: (0, 0)),
    ],
    out_specs=[
        pl.BlockSpec((_MB, HID), lambda i: (i, 0)),
        pl.BlockSpec((_MB, HID), lambda i: (i, 0)),
    ],
    out_shape=[
        jax.ShapeDtypeStruct((N, HID), jnp.float32),
        jax.ShapeDtypeStruct((N, HID), jnp.float32),
    ],
)


def _tcm2_body(acc_ref, hs_ref, dinv_ref, b_ref, w_ref, jp_ref, s_ref,
               hsn_ref):
    dinv = dinv_ref[...]
    t = dinv * (acc_ref[0] + acc_ref[1] + hs_ref[...]) + b_ref[...]
    j = jnp.maximum(t, 0.0)
    s_ref[...] = jp_ref[...] + j
    hsn_ref[...] = jnp.dot(j, w_ref[...],
                           preferred_element_type=jnp.float32) * dinv


_tcm2 = pl.pallas_call(
    _tcm2_body,
    grid=(_G,),
    in_specs=[
        pl.BlockSpec((2, _MB, HID), lambda i: (0, i, 0)),
        pl.BlockSpec((_MB, HID), lambda i: (i, 0)),
        pl.BlockSpec((_MB, 1), lambda i: (i, 0)),
        pl.BlockSpec((1, HID), lambda i: (0, 0)),
        pl.BlockSpec((HID, HID), lambda i: (0, 0)),
        pl.BlockSpec((_MB, HID), lambda i: (i, 0)),
    ],
    out_specs=[
        pl.BlockSpec((_MB, HID), lambda i: (i, 0)),
        pl.BlockSpec((_MB, HID), lambda i: (i, 0)),
    ],
    out_shape=[
        jax.ShapeDtypeStruct((N, HID), jnp.float32),
        jax.ShapeDtypeStruct((N, HID), jnp.float32),
    ],
)


def _tcf_body(acc_ref, hs_ref, dinv_ref, b_ref, s_ref, wc_ref, bc_ref,
              out_ref):
    dinv = dinv_ref[...]
    t = dinv * (acc_ref[0] + acc_ref[1] + hs_ref[...]) + b_ref[...]
    j = jnp.maximum(t, 0.0)
    x5 = s_ref[...] + j
    out_ref[...] = jnp.dot(x5, wc_ref[...],
                           preferred_element_type=jnp.float32) + bc_ref[...]


_tcf = pl.pallas_call(
    _tcf_body,
    grid=(_G,),
    in_specs=[
        pl.BlockSpec((2, _MB, HID), lambda i: (0, i, 0)),
        pl.BlockSpec((_MB, HID), lambda i: (i, 0)),
        pl.BlockSpec((_MB, 1), lambda i: (i, 0)),
        pl.BlockSpec((1, HID), lambda i: (0, 0)),
        pl.BlockSpec((_MB, HID), lambda i: (i, 0)),
        pl.BlockSpec((HID, DOUT), lambda i: (0, 0)),
        pl.BlockSpec((1, DOUT), lambda i: (0, 0)),
    ],
    out_specs=pl.BlockSpec((_MB, DOUT), lambda i: (i, 0)),
    out_shape=jax.ShapeDtypeStruct((N, DOUT), jnp.float32),
)


def kernel(x, edge_index, w1, b1, wg0, bg0, wg1, bg1, wg2, bg2, wc, bc):
    ei = edge_index.astype(jnp.int32)
    pad = EPAD - E
    ar = jnp.arange(pad, dtype=jnp.int32)
    srcp = jnp.concatenate([ei[0], (ar * 37) % N]).reshape(NW, NCH, K)
    dstp = jnp.concatenate([ei[1], N + (ar % (NPAD - N))]).reshape(NW, NCH, K)

    zrows = jnp.zeros((NPAD, HID), jnp.float32)
    orows = jnp.ones((K, HID), jnp.float32)
    degp = _deg(dstp, zrows, orows)
    dinv, hs0 = _tc1(degp, x, w1, b1.reshape(1, HID), wg0)
    acc0 = _seg(hs0, srcp, dstp, zrows)
    j1, hs1 = _tcm(acc0, hs0, dinv, bg0.reshape(1, HID), wg1)
    acc1 = _seg(hs1, srcp, dstp, zrows)
    s2, hs2 = _tcm2(acc1, hs1, dinv, bg1.reshape(1, HID), wg2, j1)
    acc2 = _seg(hs2, srcp, dstp, zrows)
    return _tcf(acc2, hs2, dinv, bg2.reshape(1, HID), s2, wc,
                bc.reshape(1, DOUT))
